# baseline (speedup 1.0000x reference)
import functools

import jax
import jax.numpy as jnp
from jax import lax
from jax.experimental import pallas as pl
from jax.experimental.pallas import tpu as pltpu
from jax.experimental.pallas import tpu_sc as plsc

N = 250000
D = 128
B = 16384
L = 16
NC = 2
NS = 16
NW = NC * NS

R = 7824
RLAST = N - (NW - 1) * R
WSZ = R + L
CH = 256
FLAT = R + CH

_mesh = plsc.VectorSubcoreMesh(
    core_axis_name="c", subcore_axis_name="s", num_cores=NC, num_subcores=NS
)


@functools.partial(
    pl.kernel,
    out_type=(
        jax.ShapeDtypeStruct((N, D), jnp.float32),
        jax.ShapeDtypeStruct((N,), jnp.int32),
    ),
    mesh=_mesh,
    compiler_params=pltpu.CompilerParams(needs_layout_passes=False),
    scratch_types=[
        pltpu.VMEM((B,), jnp.int32),
        pltpu.VMEM((WSZ,), jnp.int32),
        pltpu.VMEM((FLAT,), jnp.int32),
        pltpu.VMEM((FLAT,), jnp.int32),
        pltpu.VMEM((CH, D), jnp.float32),
        pltpu.VMEM((R,), jnp.int32),
        pltpu.SemaphoreType.DMA,
        pltpu.SemaphoreType.DMA,
    ],
)
def _sc_update(memh, idxh, bemb, bidxh, omemh, oidxh,
               bidx_v, winner_v, jflat_v, dflat_v, rowbuf_v, idxbuf_v, csem, sem):
    wid = lax.axis_index("s") * NC + lax.axis_index("c")
    lo = wid * R
    hi = jnp.minimum(lo + R, N)
    iota = lax.broadcasted_iota(jnp.int32, (L,), 0)
    is_last = wid == NW - 1

    @pl.when(jnp.logical_not(is_last))
    def _copy_full():
        pltpu.async_copy(
            memh.at[pl.ds(lo, R)], omemh.at[pl.ds(lo, R)], csem)
        pltpu.async_copy(
            idxh.at[pl.ds(lo, R)], idxbuf_v, csem)

    @pl.when(is_last)
    def _copy_tail():
        pltpu.async_copy(
            memh.at[pl.ds(lo, RLAST)], omemh.at[pl.ds(lo, RLAST)], csem)
        pltpu.async_copy(
            idxh.at[pl.ds(lo, RLAST)], idxbuf_v.at[pl.ds(0, RLAST)], csem)

    pltpu.sync_copy(bidxh, bidx_v)

    neg1 = jnp.full((L,), -1, jnp.int32)
    def _init(i, _):
        winner_v[pl.ds(i * L, L)] = neg1
        return 0
    lax.fori_loop(0, WSZ // L, _init, 0, unroll=8)

    def _scan(g, _):
        d = bidx_v[pl.ds(g * L, L)]
        j = g * L + iota
        inr = (d >= lo) & (d < hi)
        last = plsc.scan_count(d, mask=inr)[1]
        plsc.store_scatter(
            winner_v, [jnp.where(inr, d - lo, R)], j, mask=last
        )
        return 0
    lax.fori_loop(0, B // L, _scan, 0, unroll=8)

    def _compact(g, carry):
        cnt, bestv = carry
        w = winner_v[pl.ds(g * L, L)]
        m = w >= 0
        dst = lo + g * L + iota
        plsc.store_compressed(jflat_v.at[pl.ds(cnt, L)], w, mask=m)
        plsc.store_compressed(dflat_v.at[pl.ds(cnt, L)], dst, mask=m)
        popc = plsc.all_reduce_population_count(m)
        npop = popc if popc.ndim == 0 else jnp.max(popc)
        enc = jnp.where(m, (g * L + iota) * B + w, -1)
        return cnt + npop, jnp.maximum(bestv, enc)
    cnt, bestv = lax.fori_loop(
        0, R // L, _compact, (0, jnp.full((L,), -1, jnp.int32))
    )
    best = jnp.max(bestv)

    @pl.when(cnt > 0)
    def _move():
        pad_j = jnp.full((L,), best & (B - 1), jnp.int32)
        pad_d = jnp.full((L,), lo + lax.shift_right_logical(best, 14), jnp.int32)
        def _pad(t, _):
            jflat_v[pl.ds(cnt + t * L, L)] = pad_j
            dflat_v[pl.ds(cnt + t * L, L)] = pad_d
            return 0
        lax.fori_loop(0, CH // L, _pad, 0, unroll=4)

        pltpu.async_copy(
            bemb.at[jflat_v.at[pl.ds(0, CH)]], rowbuf_v, sem)

    @pl.when(jnp.logical_not(is_last))
    def _wait_full():
        pltpu.make_async_copy(
            memh.at[pl.ds(lo, R)], omemh.at[pl.ds(lo, R)], csem).wait()
        pltpu.make_async_copy(
            idxh.at[pl.ds(lo, R)], idxbuf_v, csem).wait()

    @pl.when(is_last)
    def _wait_tail():
        pltpu.make_async_copy(
            memh.at[pl.ds(lo, RLAST)], omemh.at[pl.ds(lo, RLAST)], csem).wait()
        pltpu.make_async_copy(
            idxh.at[pl.ds(lo, RLAST)], idxbuf_v.at[pl.ds(0, RLAST)], csem).wait()

    @pl.when(cnt > 0)
    def _apply():
        nch = (cnt + CH - 1) // CH

        def _chunk(ci, _):
            dlist = dflat_v.at[pl.ds(ci * CH, CH)]
            pltpu.make_async_copy(
                bemb.at[jflat_v.at[pl.ds(ci * CH, CH)]], rowbuf_v, sem
            ).wait()
            pltpu.async_copy(rowbuf_v, omemh.at[dlist], sem).wait()
            next_ci = ci + 1
            @pl.when(next_ci < nch)
            def _prefetch():
                pltpu.async_copy(
                    bemb.at[jflat_v.at[pl.ds(next_ci * CH, CH)]], rowbuf_v, sem)
            return 0
        lax.fori_loop(0, nch, _chunk, 0)

        def _iwrite(g, _):
            v = dflat_v[pl.ds(g * L, L)]
            plsc.store_scatter(idxbuf_v, [v - lo], v)
            return 0
        lax.fori_loop(0, (cnt + L - 1) // L, _iwrite, 0)

    @pl.when(jnp.logical_not(is_last))
    def _flush_full():
        pltpu.sync_copy(idxbuf_v, oidxh.at[pl.ds(lo, R)])

    @pl.when(is_last)
    def _flush_tail():
        pltpu.sync_copy(idxbuf_v.at[pl.ds(0, RLAST)], oidxh.at[pl.ds(lo, RLAST)])


def kernel(local_memory_embeddings, local_memory_index, batch_embeddings, batch_indexes):
    return _sc_update(
        local_memory_embeddings, local_memory_index,
        batch_embeddings, batch_indexes,
    )


# speedup vs baseline: 18.9938x; 18.9938x over previous
import functools

import jax
import jax.numpy as jnp
from jax import lax
from jax.experimental import pallas as pl
from jax.experimental.pallas import tpu as pltpu
from jax.experimental.pallas import tpu_sc as plsc

N = 250000
D = 128
B = 16384
L = 16
NC = 2
NS = 16
NW = NC * NS

R = 7824
WSZ = R + L
CH = 256
NCHMAX = (R + CH - 1) // CH
FLAT = NCHMAX * CH + CH

_mesh = plsc.VectorSubcoreMesh(
    core_axis_name="c", subcore_axis_name="s", num_cores=NC, num_subcores=NS
)


@functools.partial(
    pl.kernel,
    out_type=(),
    mesh=_mesh,
    compiler_params=pltpu.CompilerParams(needs_layout_passes=False),
    scratch_types=[
        pltpu.VMEM((B,), jnp.int32),
        pltpu.VMEM((WSZ,), jnp.int32),
        pltpu.VMEM((FLAT,), jnp.int32),
        pltpu.VMEM((FLAT,), jnp.int32),
        pltpu.VMEM((CH, D), jnp.float32),
        pltpu.VMEM((CH, D), jnp.float32),
        pltpu.SemaphoreType.DMA,
        pltpu.SemaphoreType.DMA,
        pltpu.SemaphoreType.DMA,
    ],
)
def _sc_update(bemb, bidxh, memh, idxh,
               bidx_v, winner_v, jflat_v, dflat_v, buf0_v, buf1_v,
               gsem, ssem, isem):
    wid = lax.axis_index("s") * NC + lax.axis_index("c")
    lo = wid * R
    hi = jnp.minimum(lo + R, N)
    iota = lax.broadcasted_iota(jnp.int32, (L,), 0)

    pltpu.sync_copy(bidxh, bidx_v)

    neg1 = jnp.full((L,), -1, jnp.int32)
    def _init(i, _):
        winner_v[pl.ds(i * L, L)] = neg1
        return 0
    lax.fori_loop(0, WSZ // L, _init, 0, unroll=8)

    def _scan(g, _):
        d = bidx_v[pl.ds(g * L, L)]
        j = g * L + iota
        inr = (d >= lo) & (d < hi)
        last = plsc.scan_count(d, mask=inr)[1]
        plsc.store_scatter(
            winner_v, [jnp.where(inr, d - lo, R)], j, mask=last
        )
        return 0
    lax.fori_loop(0, B // L, _scan, 0, unroll=8)

    def _compact(g, carry):
        cnt, bestv = carry
        w = winner_v[pl.ds(g * L, L)]
        m = w >= 0
        dst = lo + g * L + iota
        plsc.store_compressed(jflat_v.at[pl.ds(cnt, L)], w, mask=m)
        plsc.store_compressed(dflat_v.at[pl.ds(cnt, L)], dst, mask=m)
        popc = plsc.all_reduce_population_count(m)
        npop = popc if popc.ndim == 0 else jnp.max(popc)
        enc = jnp.where(m, (g * L + iota) * B + w, -1)
        return cnt + npop, jnp.maximum(bestv, enc)
    cnt, bestv = lax.fori_loop(
        0, R // L, _compact, (0, jnp.full((L,), -1, jnp.int32))
    )
    best = jnp.max(bestv)

    @pl.when(cnt > 0)
    def _move():
        pad_j = jnp.full((L,), best & (B - 1), jnp.int32)
        pad_d = jnp.full((L,), lo + lax.shift_right_logical(best, 14), jnp.int32)
        def _pad(t, _):
            jflat_v[pl.ds(cnt + t * L, L)] = pad_j
            dflat_v[pl.ds(cnt + t * L, L)] = pad_d
            return 0
        lax.fori_loop(0, CH // L, _pad, 0, unroll=4)

        nch = (cnt + CH - 1) // CH

        def _jlist(ci):
            return jflat_v.at[pl.ds(ci * CH, CH)]

        def _dlist(ci):
            return dflat_v.at[pl.ds(ci * CH, CH)]

        def _on_buf(p, fn):
            @pl.when(p == 0)
            def _a():
                fn(buf0_v)
            @pl.when(p != 0)
            def _b():
                fn(buf1_v)

        pltpu.async_copy(bemb.at[_jlist(0)], buf0_v, gsem)

        def _chunk(ci, _):
            p = ci & 1
            _on_buf(p, lambda b: pltpu.make_async_copy(
                bemb.at[_jlist(ci)], b, gsem).wait())

            @pl.when(ci + 1 < nch)
            def _next():
                @pl.when(ci >= 1)
                def _free():
                    _on_buf(1 - p, lambda b: pltpu.make_async_copy(
                        b, memh.at[_dlist(ci - 1)], ssem).wait())
                _on_buf(1 - p, lambda b: pltpu.async_copy(
                    bemb.at[_jlist(ci + 1)], b, gsem))

            _on_buf(p, lambda b: pltpu.async_copy(
                b, memh.at[_dlist(ci)], ssem))
            pltpu.async_copy(_dlist(ci), idxh.at[_dlist(ci)], isem)
            return 0
        lax.fori_loop(0, nch, _chunk, 0)

        @pl.when(nch >= 2)
        def _drain_prev():
            _on_buf((nch - 2) & 1, lambda b: pltpu.make_async_copy(
                b, memh.at[_dlist(nch - 2)], ssem).wait())
        _on_buf((nch - 1) & 1, lambda b: pltpu.make_async_copy(
            b, memh.at[_dlist(nch - 1)], ssem).wait())

        def _drain_idx(ci, _):
            pltpu.make_async_copy(_dlist(ci), idxh.at[_dlist(ci)], isem).wait()
            return 0
        lax.fori_loop(0, nch, _drain_idx, 0)


def kernel(local_memory_embeddings, local_memory_index, batch_embeddings, batch_indexes):
    mem_ref = jax.new_ref(local_memory_embeddings)
    idx_ref = jax.new_ref(local_memory_index)
    _sc_update(batch_embeddings, batch_indexes, mem_ref, idx_ref)
    return mem_ref[...], idx_ref[...]


# speedup vs baseline: 20.4663x; 1.0775x over previous
import functools

import jax
import jax.numpy as jnp
from jax import lax
from jax.experimental import pallas as pl
from jax.experimental.pallas import tpu as pltpu
from jax.experimental.pallas import tpu_sc as plsc

N = 250000
D = 128
B = 16384
L = 16
NC = 2
NS = 16
NW = NC * NS

R = 7824
RLAST = N - (NW - 1) * R
WSZ = R + L
CH = 512
FLAT = R + CH

_mesh = plsc.VectorSubcoreMesh(
    core_axis_name="c", subcore_axis_name="s", num_cores=NC, num_subcores=NS
)


@functools.partial(
    pl.kernel,
    out_type=jax.ShapeDtypeStruct((N,), jnp.int32),
    mesh=_mesh,
    compiler_params=pltpu.CompilerParams(needs_layout_passes=False),
    scratch_types=[
        pltpu.VMEM((B,), jnp.int32),
        pltpu.VMEM((WSZ,), jnp.int32),
        pltpu.VMEM((FLAT,), jnp.int32),
        pltpu.VMEM((FLAT,), jnp.int32),
        pltpu.VMEM((CH, D), jnp.float32),
        pltpu.VMEM((R,), jnp.int32),
        pltpu.SemaphoreType.DMA,
        pltpu.SemaphoreType.DMA,
    ],
)
def _sc_update(bemb, bidxh, idxh, memh, oidxh,
               bidx_v, winner_v, jflat_v, dflat_v, rowbuf_v, idxbuf_v,
               csem, sem):
    wid = lax.axis_index("s") * NC + lax.axis_index("c")
    lo = wid * R
    hi = jnp.minimum(lo + R, N)
    iota = lax.broadcasted_iota(jnp.int32, (L,), 0)
    is_last = wid == NW - 1

    @pl.when(jnp.logical_not(is_last))
    def _stage_full():
        pltpu.async_copy(idxh.at[pl.ds(lo, R)], idxbuf_v, csem)

    @pl.when(is_last)
    def _stage_tail():
        pltpu.async_copy(
            idxh.at[pl.ds(lo, RLAST)], idxbuf_v.at[pl.ds(0, RLAST)], csem)

    pltpu.sync_copy(bidxh, bidx_v)

    neg1 = jnp.full((L,), -1, jnp.int32)
    def _init(i, _):
        winner_v[pl.ds(i * L, L)] = neg1
        return 0
    lax.fori_loop(0, WSZ // L, _init, 0, unroll=8)

    def _scan(g, _):
        d = bidx_v[pl.ds(g * L, L)]
        j = g * L + iota
        inr = (d >= lo) & (d < hi)
        last = plsc.scan_count(d, mask=inr)[1]
        plsc.store_scatter(
            winner_v, [jnp.where(inr, d - lo, R)], j, mask=last
        )
        return 0
    lax.fori_loop(0, B // L, _scan, 0, unroll=8)

    def _compact(g, carry):
        cnt, bestv = carry
        w = winner_v[pl.ds(g * L, L)]
        m = w >= 0
        dst = lo + g * L + iota
        plsc.store_compressed(jflat_v.at[pl.ds(cnt, L)], w, mask=m)
        plsc.store_compressed(dflat_v.at[pl.ds(cnt, L)], dst, mask=m)
        popc = plsc.all_reduce_population_count(m)
        npop = popc if popc.ndim == 0 else jnp.max(popc)
        enc = jnp.where(m, (g * L + iota) * B + w, -1)
        return cnt + npop, jnp.maximum(bestv, enc)
    cnt, bestv = lax.fori_loop(
        0, R // L, _compact, (0, jnp.full((L,), -1, jnp.int32))
    )
    best = jnp.max(bestv)

    @pl.when(jnp.logical_not(is_last))
    def _wait_full():
        pltpu.make_async_copy(idxh.at[pl.ds(lo, R)], idxbuf_v, csem).wait()

    @pl.when(is_last)
    def _wait_tail():
        pltpu.make_async_copy(
            idxh.at[pl.ds(lo, RLAST)], idxbuf_v.at[pl.ds(0, RLAST)], csem
        ).wait()

    @pl.when(cnt > 0)
    def _move():
        pad_j = jnp.full((L,), best & (B - 1), jnp.int32)
        pad_d = jnp.full((L,), lo + lax.shift_right_logical(best, 14), jnp.int32)
        def _pad(t, _):
            jflat_v[pl.ds(cnt + t * L, L)] = pad_j
            dflat_v[pl.ds(cnt + t * L, L)] = pad_d
            return 0
        lax.fori_loop(0, CH // L, _pad, 0, unroll=4)

        def _iwrite(g, _):
            v = dflat_v[pl.ds(g * L, L)]
            plsc.store_scatter(idxbuf_v, [v - lo], v)
            return 0
        lax.fori_loop(0, (cnt + L - 1) // L, _iwrite, 0)

        nch = (cnt + CH - 1) // CH
        pltpu.async_copy(bemb.at[jflat_v.at[pl.ds(0, CH)]], rowbuf_v, sem)

        def _chunk(ci, _):
            dlist = dflat_v.at[pl.ds(ci * CH, CH)]
            pltpu.make_async_copy(
                bemb.at[jflat_v.at[pl.ds(ci * CH, CH)]], rowbuf_v, sem
            ).wait()
            pltpu.async_copy(rowbuf_v, memh.at[dlist], sem).wait()
            next_ci = ci + 1
            @pl.when(next_ci < nch)
            def _prefetch():
                pltpu.async_copy(
                    bemb.at[jflat_v.at[pl.ds(next_ci * CH, CH)]], rowbuf_v, sem)
            return 0
        lax.fori_loop(0, nch, _chunk, 0)

    @pl.when(jnp.logical_not(is_last))
    def _flush_full():
        pltpu.sync_copy(idxbuf_v, oidxh.at[pl.ds(lo, R)])

    @pl.when(is_last)
    def _flush_tail():
        pltpu.sync_copy(idxbuf_v.at[pl.ds(0, RLAST)], oidxh.at[pl.ds(lo, RLAST)])


def kernel(local_memory_embeddings, local_memory_index, batch_embeddings, batch_indexes):
    mem_ref = jax.new_ref(local_memory_embeddings)
    new_idx = _sc_update(
        batch_embeddings, batch_indexes, local_memory_index, mem_ref)
    return mem_ref[...], new_idx


# speedup vs baseline: 23.5181x; 1.1491x over previous
import functools

import jax
import jax.numpy as jnp
from jax import lax
from jax.experimental import pallas as pl
from jax.experimental.pallas import tpu as pltpu
from jax.experimental.pallas import tpu_sc as plsc

N = 250000
D = 128
B = 16384
L = 16
NC = 2
NS = 16
NW = NC * NS

R = 7824
RLAST = N - (NW - 1) * R
WSZ = R + L
CH = 256
FLAT = R + CH

_mesh = plsc.VectorSubcoreMesh(
    core_axis_name="c", subcore_axis_name="s", num_cores=NC, num_subcores=NS
)


@functools.partial(
    pl.kernel,
    out_type=jax.ShapeDtypeStruct((N,), jnp.int32),
    mesh=_mesh,
    compiler_params=pltpu.CompilerParams(needs_layout_passes=False),
    scratch_types=[
        pltpu.VMEM((B,), jnp.int32),
        pltpu.VMEM((WSZ,), jnp.int32),
        pltpu.VMEM((FLAT,), jnp.int32),
        pltpu.VMEM((FLAT,), jnp.int32),
        pltpu.VMEM((CH, D), jnp.float32),
        pltpu.VMEM((CH, D), jnp.float32),
        pltpu.VMEM((R,), jnp.int32),
        pltpu.SemaphoreType.DMA,
        pltpu.SemaphoreType.DMA,
        pltpu.SemaphoreType.DMA,
    ],
)
def _sc_update(bemb, bidxh, idxh, memh, oidxh,
               bidx_v, winner_v, jflat_v, dflat_v, buf0_v, buf1_v, idxbuf_v,
               csem, gsem, ssem):
    wid = lax.axis_index("s") * NC + lax.axis_index("c")
    lo = wid * R
    hi = jnp.minimum(lo + R, N)
    iota = lax.broadcasted_iota(jnp.int32, (L,), 0)
    is_last = wid == NW - 1

    @pl.when(jnp.logical_not(is_last))
    def _stage_full():
        pltpu.async_copy(idxh.at[pl.ds(lo, R)], idxbuf_v, csem)

    @pl.when(is_last)
    def _stage_tail():
        pltpu.async_copy(
            idxh.at[pl.ds(lo, RLAST)], idxbuf_v.at[pl.ds(0, RLAST)], csem)

    pltpu.sync_copy(bidxh, bidx_v)

    neg1 = jnp.full((L,), -1, jnp.int32)
    def _init(i, _):
        winner_v[pl.ds(i * L, L)] = neg1
        return 0
    lax.fori_loop(0, WSZ // L, _init, 0, unroll=8)

    def _scan(g, _):
        d = bidx_v[pl.ds(g * L, L)]
        j = g * L + iota
        inr = (d >= lo) & (d < hi)
        last = plsc.scan_count(d, mask=inr)[1]
        plsc.store_scatter(
            winner_v, [jnp.where(inr, d - lo, R)], j, mask=last
        )
        return 0
    lax.fori_loop(0, B // L, _scan, 0, unroll=8)

    def _jlist(ci):
        return jflat_v.at[pl.ds(ci * CH, CH)]

    def _dlist(ci):
        return dflat_v.at[pl.ds(ci * CH, CH)]

    def _on_buf(p, fn):
        @pl.when(p == 0)
        def _a():
            fn(buf0_v)
        @pl.when(p != 0)
        def _b():
            fn(buf1_v)

    def _compact(g, carry):
        cnt, bestv, fired = carry
        w = winner_v[pl.ds(g * L, L)]
        m = w >= 0
        dst = lo + g * L + iota
        plsc.store_compressed(jflat_v.at[pl.ds(cnt, L)], w, mask=m)
        plsc.store_compressed(dflat_v.at[pl.ds(cnt, L)], dst, mask=m)
        popc = plsc.all_reduce_population_count(m)
        npop = popc if popc.ndim == 0 else jnp.max(popc)
        ncnt = cnt + npop
        can_fire = ((fired + 1) * CH <= ncnt) & (fired < 2)
        @pl.when(can_fire)
        def _fire():
            _on_buf(fired & 1, lambda b: pltpu.async_copy(
                bemb.at[jflat_v.at[pl.ds(fired * CH, CH)]], b, gsem))
        enc = jnp.where(m, (g * L + iota) * B + w, -1)
        return ncnt, jnp.maximum(bestv, enc), fired + can_fire.astype(jnp.int32)
    cnt, bestv, nfired = lax.fori_loop(
        0, R // L, _compact, (0, jnp.full((L,), -1, jnp.int32), 0)
    )
    best = jnp.max(bestv)

    @pl.when(jnp.logical_not(is_last))
    def _wait_full():
        pltpu.make_async_copy(idxh.at[pl.ds(lo, R)], idxbuf_v, csem).wait()

    @pl.when(is_last)
    def _wait_tail():
        pltpu.make_async_copy(
            idxh.at[pl.ds(lo, RLAST)], idxbuf_v.at[pl.ds(0, RLAST)], csem
        ).wait()

    @pl.when(cnt > 0)
    def _move():
        pad_j = jnp.full((L,), best & (B - 1), jnp.int32)
        pad_d = jnp.full((L,), lo + lax.shift_right_logical(best, 14), jnp.int32)
        def _pad(t, _):
            jflat_v[pl.ds(cnt + t * L, L)] = pad_j
            dflat_v[pl.ds(cnt + t * L, L)] = pad_d
            return 0
        lax.fori_loop(0, CH // L, _pad, 0, unroll=4)

        def _iwrite(g, _):
            v = dflat_v[pl.ds(g * L, L)]
            plsc.store_scatter(idxbuf_v, [v - lo], v)
            return 0
        lax.fori_loop(0, (cnt + L - 1) // L, _iwrite, 0)

        nch = (cnt + CH - 1) // CH

        @pl.when(nfired == 0)
        def _fire0():
            pltpu.async_copy(bemb.at[_jlist(0)], buf0_v, gsem)

        def _chunk(ci, _):
            p = ci & 1
            _on_buf(p, lambda b: pltpu.make_async_copy(
                bemb.at[_jlist(ci)], b, gsem).wait())

            @pl.when(ci + 1 < nch)
            def _next():
                @pl.when(ci >= 1)
                def _free():
                    _on_buf(1 - p, lambda b: pltpu.make_async_copy(
                        b, memh.at[_dlist(ci - 1)], ssem).wait())
                @pl.when(ci + 1 >= jnp.maximum(nfired, 1))
                def _fire():
                    _on_buf(1 - p, lambda b: pltpu.async_copy(
                        bemb.at[_jlist(ci + 1)], b, gsem))

            _on_buf(p, lambda b: pltpu.async_copy(
                b, memh.at[_dlist(ci)], ssem))
            return 0
        lax.fori_loop(0, nch, _chunk, 0)

        @pl.when(nch >= 2)
        def _drain_prev():
            _on_buf((nch - 2) & 1, lambda b: pltpu.make_async_copy(
                b, memh.at[_dlist(nch - 2)], ssem).wait())
        _on_buf((nch - 1) & 1, lambda b: pltpu.make_async_copy(
            b, memh.at[_dlist(nch - 1)], ssem).wait())

    @pl.when(jnp.logical_not(is_last))
    def _flush_full():
        pltpu.sync_copy(idxbuf_v, oidxh.at[pl.ds(lo, R)])

    @pl.when(is_last)
    def _flush_tail():
        pltpu.sync_copy(idxbuf_v.at[pl.ds(0, RLAST)], oidxh.at[pl.ds(lo, RLAST)])


def kernel(local_memory_embeddings, local_memory_index, batch_embeddings, batch_indexes):
    mem_ref = jax.new_ref(local_memory_embeddings)
    new_idx = _sc_update(
        batch_embeddings, batch_indexes, local_memory_index, mem_ref)
    return mem_ref[...], new_idx


# speedup vs baseline: 23.9214x; 1.0171x over previous
import functools

import jax
import jax.numpy as jnp
from jax import lax
from jax.experimental import pallas as pl
from jax.experimental.pallas import tpu as pltpu
from jax.experimental.pallas import tpu_sc as plsc

N = 250000
D = 128
B = 16384
L = 16
NC = 2
NS = 16
NW = NC * NS

R = 7824
RLAST = N - (NW - 1) * R
WSZ = R + L
CH = 256
FLAT = R + CH

_mesh = plsc.VectorSubcoreMesh(
    core_axis_name="c", subcore_axis_name="s", num_cores=NC, num_subcores=NS
)


@functools.partial(
    pl.kernel,
    out_type=jax.ShapeDtypeStruct((N,), jnp.int32),
    mesh=_mesh,
    compiler_params=pltpu.CompilerParams(needs_layout_passes=False),
    scratch_types=[
        pltpu.VMEM((B,), jnp.int32),
        pltpu.VMEM((WSZ,), jnp.int32),
        pltpu.VMEM((FLAT,), jnp.int32),
        pltpu.VMEM((FLAT,), jnp.int32),
        pltpu.VMEM((CH, D), jnp.float32),
        pltpu.VMEM((CH, D), jnp.float32),
        pltpu.VMEM((R,), jnp.int32),
        pltpu.SemaphoreType.DMA,
        pltpu.SemaphoreType.DMA,
        pltpu.SemaphoreType.DMA,
    ],
)
def _sc_update(bemb, bidxh, idxh, memh, oidxh,
               bidx_v, winner_v, jflat_v, dflat_v, buf0_v, buf1_v, idxbuf_v,
               csem, gsem, ssem):
    wid = lax.axis_index("s") * NC + lax.axis_index("c")
    lo = wid * R
    hi = jnp.minimum(lo + R, N)
    iota = lax.broadcasted_iota(jnp.int32, (L,), 0)
    is_last = wid == NW - 1

    @pl.when(jnp.logical_not(is_last))
    def _stage_full():
        pltpu.async_copy(idxh.at[pl.ds(lo, R)], idxbuf_v, csem)

    @pl.when(is_last)
    def _stage_tail():
        pltpu.async_copy(
            idxh.at[pl.ds(lo, RLAST)], idxbuf_v.at[pl.ds(0, RLAST)], csem)

    pltpu.sync_copy(bidxh, bidx_v)

    neg1 = jnp.full((L,), -1, jnp.int32)
    def _init(i, _):
        winner_v[pl.ds(i * L, L)] = neg1
        return 0
    lax.fori_loop(0, WSZ // L, _init, 0, unroll=8)

    def _scan(g, _):
        d = bidx_v[pl.ds(g * L, L)]
        j = g * L + iota
        inr = (d >= lo) & (d < hi)
        last = plsc.scan_count(d, mask=inr)[1]
        plsc.store_scatter(
            winner_v, [jnp.where(inr, d - lo, R)], j, mask=last
        )
        return 0
    lax.fori_loop(0, B // L, _scan, 0, unroll=8)

    def _jlist(ci):
        return jflat_v.at[pl.ds(ci * CH, CH)]

    def _dlist(ci):
        return dflat_v.at[pl.ds(ci * CH, CH)]

    def _on_buf(p, fn):
        @pl.when(p == 0)
        def _a():
            fn(buf0_v)
        @pl.when(p != 0)
        def _b():
            fn(buf1_v)

    def _compact(g, carry):
        cnt, bestv, fired = carry
        w = winner_v[pl.ds(g * L, L)]
        m = w >= 0
        dst = lo + g * L + iota
        plsc.store_compressed(jflat_v.at[pl.ds(cnt, L)], w, mask=m)
        plsc.store_compressed(dflat_v.at[pl.ds(cnt, L)], dst, mask=m)
        popc = plsc.all_reduce_population_count(m)
        npop = popc if popc.ndim == 0 else lax.squeeze(
            lax.slice(popc, (0,), (1,)), (0,))
        ncnt = cnt + npop
        can_fire = ((fired + 1) * CH <= ncnt) & (fired < 2)
        @pl.when(can_fire)
        def _fire():
            _on_buf(fired & 1, lambda b: pltpu.async_copy(
                bemb.at[jflat_v.at[pl.ds(fired * CH, CH)]], b, gsem))
        enc = jnp.where(m, (g * L + iota) * B + w, -1)
        return ncnt, jnp.maximum(bestv, enc), fired + can_fire.astype(jnp.int32)
    cnt, bestv, nfired = lax.fori_loop(
        0, R // L, _compact, (0, jnp.full((L,), -1, jnp.int32), 0)
    )
    best = jnp.max(bestv)

    @pl.when(jnp.logical_not(is_last))
    def _wait_full():
        pltpu.make_async_copy(idxh.at[pl.ds(lo, R)], idxbuf_v, csem).wait()

    @pl.when(is_last)
    def _wait_tail():
        pltpu.make_async_copy(
            idxh.at[pl.ds(lo, RLAST)], idxbuf_v.at[pl.ds(0, RLAST)], csem
        ).wait()

    @pl.when(cnt > 0)
    def _move():
        pad_j = jnp.full((L,), best & (B - 1), jnp.int32)
        pad_d = jnp.full((L,), lo + lax.shift_right_logical(best, 14), jnp.int32)
        def _pad(t, _):
            jflat_v[pl.ds(cnt + t * L, L)] = pad_j
            dflat_v[pl.ds(cnt + t * L, L)] = pad_d
            return 0
        lax.fori_loop(0, CH // L, _pad, 0, unroll=4)

        def _iwrite(g, _):
            v = dflat_v[pl.ds(g * L, L)]
            plsc.store_scatter(idxbuf_v, [v - lo], v)
            return 0
        lax.fori_loop(0, (cnt + L - 1) // L, _iwrite, 0)

        @pl.when(jnp.logical_not(is_last))
        def _enq_full():
            pltpu.async_copy(idxbuf_v, oidxh.at[pl.ds(lo, R)], csem)

        @pl.when(is_last)
        def _enq_tail():
            pltpu.async_copy(
                idxbuf_v.at[pl.ds(0, RLAST)], oidxh.at[pl.ds(lo, RLAST)], csem)

        nch = (cnt + CH - 1) // CH

        @pl.when(nfired == 0)
        def _fire0():
            pltpu.async_copy(bemb.at[_jlist(0)], buf0_v, gsem)

        def _chunk(ci, _):
            p = ci & 1
            _on_buf(p, lambda b: pltpu.make_async_copy(
                bemb.at[_jlist(ci)], b, gsem).wait())

            @pl.when(ci + 1 < nch)
            def _next():
                @pl.when(ci >= 1)
                def _free():
                    _on_buf(1 - p, lambda b: pltpu.make_async_copy(
                        b, memh.at[_dlist(ci - 1)], ssem).wait())
                @pl.when(ci + 1 >= jnp.maximum(nfired, 1))
                def _fire():
                    _on_buf(1 - p, lambda b: pltpu.async_copy(
                        bemb.at[_jlist(ci + 1)], b, gsem))

            _on_buf(p, lambda b: pltpu.async_copy(
                b, memh.at[_dlist(ci)], ssem))
            return 0
        lax.fori_loop(0, nch, _chunk, 0)

        @pl.when(nch >= 2)
        def _drain_prev():
            _on_buf((nch - 2) & 1, lambda b: pltpu.make_async_copy(
                b, memh.at[_dlist(nch - 2)], ssem).wait())
        _on_buf((nch - 1) & 1, lambda b: pltpu.make_async_copy(
            b, memh.at[_dlist(nch - 1)], ssem).wait())

    @pl.when(cnt == 0)
    def _enq_empty():
        @pl.when(jnp.logical_not(is_last))
        def _e_full():
            pltpu.async_copy(idxbuf_v, oidxh.at[pl.ds(lo, R)], csem)

        @pl.when(is_last)
        def _e_tail():
            pltpu.async_copy(
                idxbuf_v.at[pl.ds(0, RLAST)], oidxh.at[pl.ds(lo, RLAST)], csem)

    @pl.when(jnp.logical_not(is_last))
    def _flush_full():
        pltpu.make_async_copy(idxbuf_v, oidxh.at[pl.ds(lo, R)], csem).wait()

    @pl.when(is_last)
    def _flush_tail():
        pltpu.make_async_copy(
            idxbuf_v.at[pl.ds(0, RLAST)], oidxh.at[pl.ds(lo, RLAST)], csem
        ).wait()


def kernel(local_memory_embeddings, local_memory_index, batch_embeddings, batch_indexes):
    mem_ref = jax.new_ref(local_memory_embeddings)
    new_idx = _sc_update(
        batch_embeddings, batch_indexes, local_memory_index, mem_ref)
    return mem_ref[...], new_idx


# speedup vs baseline: 26.2346x; 1.0967x over previous
import functools

import jax
import jax.numpy as jnp
from jax import lax
from jax.experimental import pallas as pl
from jax.experimental.pallas import tpu as pltpu
from jax.experimental.pallas import tpu_sc as plsc

N = 250000
D = 128
B = 16384
L = 16
NC = 2
NS = 16
NW = NC * NS

R = 7824
RLAST = N - (NW - 1) * R
WSZ = R + L
CH = 128
NBUF = 4
FLAT = R + CH

_mesh = plsc.VectorSubcoreMesh(
    core_axis_name="c", subcore_axis_name="s", num_cores=NC, num_subcores=NS
)


@functools.partial(
    pl.kernel,
    out_type=jax.ShapeDtypeStruct((N,), jnp.int32),
    mesh=_mesh,
    compiler_params=pltpu.CompilerParams(needs_layout_passes=False),
    scratch_types=[
        pltpu.VMEM((B,), jnp.int32),
        pltpu.VMEM((WSZ,), jnp.int32),
        pltpu.VMEM((FLAT,), jnp.int32),
        pltpu.VMEM((FLAT,), jnp.int32),
        pltpu.VMEM((CH, D), jnp.float32),
        pltpu.VMEM((CH, D), jnp.float32),
        pltpu.VMEM((CH, D), jnp.float32),
        pltpu.VMEM((CH, D), jnp.float32),
        pltpu.VMEM((R,), jnp.int32),
        pltpu.SemaphoreType.DMA,
        pltpu.SemaphoreType.DMA,
        pltpu.SemaphoreType.DMA,
    ],
)
def _sc_update(bemb, bidxh, idxh, memh, oidxh,
               bidx_v, winner_v, jflat_v, dflat_v,
               buf0_v, buf1_v, buf2_v, buf3_v, idxbuf_v,
               csem, gsem, ssem):
    wid = lax.axis_index("s") * NC + lax.axis_index("c")
    lo = wid * R
    hi = jnp.minimum(lo + R, N)
    iota = lax.broadcasted_iota(jnp.int32, (L,), 0)
    is_last = wid == NW - 1

    @pl.when(jnp.logical_not(is_last))
    def _stage_full():
        pltpu.async_copy(idxh.at[pl.ds(lo, R)], idxbuf_v, csem)

    @pl.when(is_last)
    def _stage_tail():
        pltpu.async_copy(
            idxh.at[pl.ds(lo, RLAST)], idxbuf_v.at[pl.ds(0, RLAST)], csem)

    pltpu.sync_copy(bidxh, bidx_v)

    neg1 = jnp.full((L,), -1, jnp.int32)
    def _init(i, _):
        winner_v[pl.ds(i * L, L)] = neg1
        return 0
    lax.fori_loop(0, WSZ // L, _init, 0, unroll=8)

    def _scan(g, _):
        d = bidx_v[pl.ds(g * L, L)]
        j = g * L + iota
        inr = (d >= lo) & (d < hi)
        last = plsc.scan_count(d, mask=inr)[1]
        plsc.store_scatter(
            winner_v, [jnp.where(inr, d - lo, R)], j, mask=last
        )
        return 0
    lax.fori_loop(0, B // L, _scan, 0, unroll=8)

    def _jlist(ci):
        return jflat_v.at[pl.ds(ci * CH, CH)]

    def _dlist(ci):
        return dflat_v.at[pl.ds(ci * CH, CH)]

    def _on_buf(p, fn):
        @pl.when(p == 0)
        def _a():
            fn(buf0_v)
        @pl.when(p == 1)
        def _b():
            fn(buf1_v)
        @pl.when(p == 2)
        def _c():
            fn(buf2_v)
        @pl.when(p == 3)
        def _d():
            fn(buf3_v)

    def _compact(g, carry):
        cnt, bestv, fired = carry
        w = winner_v[pl.ds(g * L, L)]
        m = w >= 0
        dst = lo + g * L + iota
        plsc.store_compressed(jflat_v.at[pl.ds(cnt, L)], w, mask=m)
        plsc.store_compressed(dflat_v.at[pl.ds(cnt, L)], dst, mask=m)
        popc = plsc.all_reduce_population_count(m)
        npop = popc if popc.ndim == 0 else lax.squeeze(
            lax.slice(popc, (0,), (1,)), (0,))
        ncnt = cnt + npop
        can_fire = ((fired + 1) * CH <= ncnt) & (fired < NBUF)
        @pl.when(can_fire)
        def _fire():
            _on_buf(fired & (NBUF - 1), lambda b: pltpu.async_copy(
                bemb.at[jflat_v.at[pl.ds(fired * CH, CH)]], b, gsem))
        enc = jnp.where(m, (g * L + iota) * B + w, -1)
        return ncnt, jnp.maximum(bestv, enc), fired + can_fire.astype(jnp.int32)
    cnt, bestv, nfired = lax.fori_loop(
        0, R // L, _compact, (0, jnp.full((L,), -1, jnp.int32), 0)
    )
    best = jnp.max(bestv)

    @pl.when(jnp.logical_not(is_last))
    def _wait_full():
        pltpu.make_async_copy(idxh.at[pl.ds(lo, R)], idxbuf_v, csem).wait()

    @pl.when(is_last)
    def _wait_tail():
        pltpu.make_async_copy(
            idxh.at[pl.ds(lo, RLAST)], idxbuf_v.at[pl.ds(0, RLAST)], csem
        ).wait()

    @pl.when(cnt > 0)
    def _move():
        pad_j = jnp.full((L,), best & (B - 1), jnp.int32)
        pad_d = jnp.full((L,), lo + lax.shift_right_logical(best, 14), jnp.int32)
        def _pad(t, _):
            jflat_v[pl.ds(cnt + t * L, L)] = pad_j
            dflat_v[pl.ds(cnt + t * L, L)] = pad_d
            return 0
        lax.fori_loop(0, CH // L, _pad, 0, unroll=4)

        def _iwrite(g, _):
            v = dflat_v[pl.ds(g * L, L)]
            plsc.store_scatter(idxbuf_v, [v - lo], v)
            return 0
        lax.fori_loop(0, (cnt + L - 1) // L, _iwrite, 0)

        @pl.when(jnp.logical_not(is_last))
        def _enq_full():
            pltpu.async_copy(idxbuf_v, oidxh.at[pl.ds(lo, R)], csem)

        @pl.when(is_last)
        def _enq_tail():
            pltpu.async_copy(
                idxbuf_v.at[pl.ds(0, RLAST)], oidxh.at[pl.ds(lo, RLAST)], csem)

        nch = (cnt + CH - 1) // CH

        @pl.when(nfired == 0)
        def _fire0():
            pltpu.async_copy(bemb.at[_jlist(0)], buf0_v, gsem)

        def _chunk(ci, _):
            p = ci & (NBUF - 1)
            _on_buf(p, lambda b: pltpu.make_async_copy(
                bemb.at[_jlist(ci)], b, gsem).wait())

            @pl.when(ci + 1 < nch)
            def _next():
                q = (ci + 1) & (NBUF - 1)
                @pl.when(ci >= NBUF - 1)
                def _free():
                    _on_buf(q, lambda b: pltpu.make_async_copy(
                        b, memh.at[_dlist(ci + 1 - NBUF)], ssem).wait())
                @pl.when(ci + 1 >= jnp.maximum(nfired, 1))
                def _fire():
                    _on_buf(q, lambda b: pltpu.async_copy(
                        bemb.at[_jlist(ci + 1)], b, gsem))

            _on_buf(p, lambda b: pltpu.async_copy(
                b, memh.at[_dlist(ci)], ssem))
            return 0
        lax.fori_loop(0, nch, _chunk, 0)

        def _drain(k, _):
            ci = jnp.maximum(nch - NBUF, 0) + k
            @pl.when(ci < nch)
            def _w():
                _on_buf(ci & (NBUF - 1), lambda b: pltpu.make_async_copy(
                    b, memh.at[_dlist(ci)], ssem).wait())
            return 0
        lax.fori_loop(0, jnp.minimum(nch, NBUF), _drain, 0)

    @pl.when(cnt == 0)
    def _enq_empty():
        @pl.when(jnp.logical_not(is_last))
        def _e_full():
            pltpu.async_copy(idxbuf_v, oidxh.at[pl.ds(lo, R)], csem)

        @pl.when(is_last)
        def _e_tail():
            pltpu.async_copy(
                idxbuf_v.at[pl.ds(0, RLAST)], oidxh.at[pl.ds(lo, RLAST)], csem)

    @pl.when(jnp.logical_not(is_last))
    def _flush_full():
        pltpu.make_async_copy(idxbuf_v, oidxh.at[pl.ds(lo, R)], csem).wait()

    @pl.when(is_last)
    def _flush_tail():
        pltpu.make_async_copy(
            idxbuf_v.at[pl.ds(0, RLAST)], oidxh.at[pl.ds(lo, RLAST)], csem
        ).wait()


def kernel(local_memory_embeddings, local_memory_index, batch_embeddings, batch_indexes):
    mem_ref = jax.new_ref(local_memory_embeddings)
    new_idx = _sc_update(
        batch_embeddings, batch_indexes, local_memory_index, mem_ref)
    return mem_ref[...], new_idx


# speedup vs baseline: 26.7817x; 1.0209x over previous
import functools

import jax
import jax.numpy as jnp
from jax import lax
from jax.experimental import pallas as pl
from jax.experimental.pallas import tpu as pltpu
from jax.experimental.pallas import tpu_sc as plsc

N = 250000
D = 128
B = 16384
L = 16
NC = 2
NS = 16
NW = NC * NS

R = 7824
RLAST = N - (NW - 1) * R
WSZ = R + L
CH = 96
NBUF = 6
FLAT = R + CH

_mesh = plsc.VectorSubcoreMesh(
    core_axis_name="c", subcore_axis_name="s", num_cores=NC, num_subcores=NS
)


@functools.partial(
    pl.kernel,
    out_type=jax.ShapeDtypeStruct((N,), jnp.int32),
    mesh=_mesh,
    compiler_params=pltpu.CompilerParams(needs_layout_passes=False),
    scratch_types=[
        pltpu.VMEM((B,), jnp.int32),
        pltpu.VMEM((WSZ,), jnp.int32),
        pltpu.VMEM((FLAT,), jnp.int32),
        pltpu.VMEM((FLAT,), jnp.int32),
        pltpu.VMEM((CH, D), jnp.float32),
        pltpu.VMEM((CH, D), jnp.float32),
        pltpu.VMEM((CH, D), jnp.float32),
        pltpu.VMEM((CH, D), jnp.float32),
        pltpu.VMEM((CH, D), jnp.float32),
        pltpu.VMEM((CH, D), jnp.float32),
        pltpu.VMEM((R,), jnp.int32),
        pltpu.SemaphoreType.DMA,
        pltpu.SemaphoreType.DMA,
        pltpu.SemaphoreType.DMA,
    ],
)
def _sc_update(bemb, bidxh, idxh, memh, oidxh,
               bidx_v, winner_v, jflat_v, dflat_v,
               buf0_v, buf1_v, buf2_v, buf3_v, buf4_v, buf5_v, idxbuf_v,
               csem, gsem, ssem):
    wid = lax.axis_index("s") * NC + lax.axis_index("c")
    lo = wid * R
    hi = jnp.minimum(lo + R, N)
    iota = lax.broadcasted_iota(jnp.int32, (L,), 0)
    is_last = wid == NW - 1

    @pl.when(jnp.logical_not(is_last))
    def _stage_full():
        pltpu.async_copy(idxh.at[pl.ds(lo, R)], idxbuf_v, csem)

    @pl.when(is_last)
    def _stage_tail():
        pltpu.async_copy(
            idxh.at[pl.ds(lo, RLAST)], idxbuf_v.at[pl.ds(0, RLAST)], csem)

    pltpu.sync_copy(bidxh, bidx_v)

    neg1 = jnp.full((L,), -1, jnp.int32)
    def _init(i, _):
        winner_v[pl.ds(i * L, L)] = neg1
        return 0
    lax.fori_loop(0, WSZ // L, _init, 0, unroll=8)

    def _scan(g, _):
        d = bidx_v[pl.ds(g * L, L)]
        j = g * L + iota
        inr = (d >= lo) & (d < hi)
        last = plsc.scan_count(d, mask=inr)[1]
        plsc.store_scatter(
            winner_v, [jnp.where(inr, d - lo, R)], j, mask=last
        )
        return 0
    lax.fori_loop(0, B // L, _scan, 0, unroll=8)

    def _jlist(ci):
        return jflat_v.at[pl.ds(ci * CH, CH)]

    def _dlist(ci):
        return dflat_v.at[pl.ds(ci * CH, CH)]

    def _on_buf(p, fn):
        @pl.when(p == 0)
        def _a():
            fn(buf0_v)
        @pl.when(p == 1)
        def _b():
            fn(buf1_v)
        @pl.when(p == 2)
        def _c():
            fn(buf2_v)
        @pl.when(p == 3)
        def _d():
            fn(buf3_v)
        @pl.when(p == 4)
        def _e():
            fn(buf4_v)
        @pl.when(p == 5)
        def _f():
            fn(buf5_v)

    def _compact(g, carry):
        cnt, bestv, fired = carry
        w = winner_v[pl.ds(g * L, L)]
        m = w >= 0
        dst = lo + g * L + iota
        plsc.store_compressed(jflat_v.at[pl.ds(cnt, L)], w, mask=m)
        plsc.store_compressed(dflat_v.at[pl.ds(cnt, L)], dst, mask=m)
        popc = plsc.all_reduce_population_count(m)
        npop = popc if popc.ndim == 0 else lax.squeeze(
            lax.slice(popc, (0,), (1,)), (0,))
        ncnt = cnt + npop
        can_fire = ((fired + 1) * CH <= ncnt) & (fired < NBUF)
        @pl.when(can_fire)
        def _fire():
            _on_buf(lax.rem(fired, NBUF), lambda b: pltpu.async_copy(
                bemb.at[jflat_v.at[pl.ds(fired * CH, CH)]], b, gsem))
        enc = jnp.where(m, (g * L + iota) * B + w, -1)
        return ncnt, jnp.maximum(bestv, enc), fired + can_fire.astype(jnp.int32)
    cnt, bestv, nfired = lax.fori_loop(
        0, R // L, _compact, (0, jnp.full((L,), -1, jnp.int32), 0)
    )
    best = jnp.max(bestv)

    @pl.when(jnp.logical_not(is_last))
    def _wait_full():
        pltpu.make_async_copy(idxh.at[pl.ds(lo, R)], idxbuf_v, csem).wait()

    @pl.when(is_last)
    def _wait_tail():
        pltpu.make_async_copy(
            idxh.at[pl.ds(lo, RLAST)], idxbuf_v.at[pl.ds(0, RLAST)], csem
        ).wait()

    @pl.when(cnt > 0)
    def _move():
        pad_j = jnp.full((L,), best & (B - 1), jnp.int32)
        pad_d = jnp.full((L,), lo + lax.shift_right_logical(best, 14), jnp.int32)
        def _pad(t, _):
            jflat_v[pl.ds(cnt + t * L, L)] = pad_j
            dflat_v[pl.ds(cnt + t * L, L)] = pad_d
            return 0
        lax.fori_loop(0, CH // L, _pad, 0, unroll=4)

        def _iwrite(g, _):
            v = dflat_v[pl.ds(g * L, L)]
            plsc.store_scatter(idxbuf_v, [v - lo], v)
            return 0
        lax.fori_loop(0, (cnt + L - 1) // L, _iwrite, 0)

        @pl.when(jnp.logical_not(is_last))
        def _enq_full():
            pltpu.async_copy(idxbuf_v, oidxh.at[pl.ds(lo, R)], csem)

        @pl.when(is_last)
        def _enq_tail():
            pltpu.async_copy(
                idxbuf_v.at[pl.ds(0, RLAST)], oidxh.at[pl.ds(lo, RLAST)], csem)

        nch = (cnt + CH - 1) // CH

        @pl.when(nfired == 0)
        def _fire0():
            pltpu.async_copy(bemb.at[_jlist(0)], buf0_v, gsem)

        def _chunk(ci, _):
            p = lax.rem(ci, NBUF)
            _on_buf(p, lambda b: pltpu.make_async_copy(
                bemb.at[_jlist(ci)], b, gsem).wait())

            @pl.when(ci + 1 < nch)
            def _next():
                q = lax.rem(ci + 1, NBUF)
                @pl.when(ci >= NBUF - 1)
                def _free():
                    _on_buf(q, lambda b: pltpu.make_async_copy(
                        b, memh.at[_dlist(ci + 1 - NBUF)], ssem).wait())
                @pl.when(ci + 1 >= jnp.maximum(nfired, 1))
                def _fire():
                    _on_buf(q, lambda b: pltpu.async_copy(
                        bemb.at[_jlist(ci + 1)], b, gsem))

            _on_buf(p, lambda b: pltpu.async_copy(
                b, memh.at[_dlist(ci)], ssem))
            return 0
        lax.fori_loop(0, nch, _chunk, 0)

        def _drain(k, _):
            ci = jnp.maximum(nch - NBUF, 0) + k
            @pl.when(ci < nch)
            def _w():
                _on_buf(lax.rem(ci, NBUF), lambda b: pltpu.make_async_copy(
                    b, memh.at[_dlist(ci)], ssem).wait())
            return 0
        lax.fori_loop(0, jnp.minimum(nch, NBUF), _drain, 0)

    @pl.when(cnt == 0)
    def _enq_empty():
        @pl.when(jnp.logical_not(is_last))
        def _e_full():
            pltpu.async_copy(idxbuf_v, oidxh.at[pl.ds(lo, R)], csem)

        @pl.when(is_last)
        def _e_tail():
            pltpu.async_copy(
                idxbuf_v.at[pl.ds(0, RLAST)], oidxh.at[pl.ds(lo, RLAST)], csem)

    @pl.when(jnp.logical_not(is_last))
    def _flush_full():
        pltpu.make_async_copy(idxbuf_v, oidxh.at[pl.ds(lo, R)], csem).wait()

    @pl.when(is_last)
    def _flush_tail():
        pltpu.make_async_copy(
            idxbuf_v.at[pl.ds(0, RLAST)], oidxh.at[pl.ds(lo, RLAST)], csem
        ).wait()


def kernel(local_memory_embeddings, local_memory_index, batch_embeddings, batch_indexes):
    mem_ref = jax.new_ref(local_memory_embeddings)
    new_idx = _sc_update(
        batch_embeddings, batch_indexes, local_memory_index, mem_ref)
    return mem_ref[...], new_idx


# speedup vs baseline: 28.1713x; 1.0519x over previous
import functools

import jax
import jax.numpy as jnp
from jax import lax
from jax.experimental import pallas as pl
from jax.experimental.pallas import tpu as pltpu
from jax.experimental.pallas import tpu_sc as plsc

N = 250000
D = 128
B = 16384
L = 16
NC = 2
NS = 16
NW = NC * NS

R = 7824
RLAST = N - (NW - 1) * R
WSZ = R + L
CH = 96
NAB = 2
FLAT = R + CH

CC = 48
NCK = R // CC
NCKL = RLAST // CC
CTAIL = RLAST - NCKL * CC
NCB = 6

_mesh = plsc.VectorSubcoreMesh(
    core_axis_name="c", subcore_axis_name="s", num_cores=NC, num_subcores=NS
)


@functools.partial(
    pl.kernel,
    out_type=(
        jax.ShapeDtypeStruct((N, D), jnp.float32),
        jax.ShapeDtypeStruct((N,), jnp.int32),
    ),
    mesh=_mesh,
    compiler_params=pltpu.CompilerParams(needs_layout_passes=False),
    scratch_types=[
        pltpu.VMEM((B,), jnp.int32),
        pltpu.VMEM((WSZ,), jnp.int32),
        pltpu.VMEM((FLAT,), jnp.int32),
        pltpu.VMEM((FLAT,), jnp.int32),
        pltpu.VMEM((CH, D), jnp.float32),
        pltpu.VMEM((CH, D), jnp.float32),
        pltpu.VMEM((CC, D), jnp.float32),
        pltpu.VMEM((CC, D), jnp.float32),
        pltpu.VMEM((CC, D), jnp.float32),
        pltpu.VMEM((CC, D), jnp.float32),
        pltpu.VMEM((CC, D), jnp.float32),
        pltpu.VMEM((CC, D), jnp.float32),
        pltpu.VMEM((R,), jnp.int32),
        pltpu.SemaphoreType.DMA,
        pltpu.SemaphoreType.DMA,
        pltpu.SemaphoreType.DMA,
        pltpu.SemaphoreType.DMA,
        pltpu.SemaphoreType.DMA,
    ],
)
def _sc_update(memh, idxh, bemb, bidxh, omemh, oidxh,
               bidx_v, winner_v, jflat_v, dflat_v, abuf0_v, abuf1_v,
               cbuf0_v, cbuf1_v, cbuf2_v, cbuf3_v, cbuf4_v, cbuf5_v, idxbuf_v,
               isem, cisem, cosem, gsem, ssem):
    wid = lax.axis_index("s") * NC + lax.axis_index("c")
    lo = wid * R
    hi = jnp.minimum(lo + R, N)
    iota = lax.broadcasted_iota(jnp.int32, (L,), 0)
    is_last = wid == NW - 1
    nck = jnp.where(is_last, NCKL, NCK)

    def _src(k):
        return memh.at[pl.ds(lo + k * CC, CC)]

    def _dst(k):
        return omemh.at[pl.ds(lo + k * CC, CC)]

    def _on_cbuf(p, fn):
        @pl.when(p == 0)
        def _a():
            fn(cbuf0_v)
        @pl.when(p == 1)
        def _b():
            fn(cbuf1_v)
        @pl.when(p == 2)
        def _c():
            fn(cbuf2_v)
        @pl.when(p == 3)
        def _d():
            fn(cbuf3_v)
        @pl.when(p == 4)
        def _e():
            fn(cbuf4_v)
        @pl.when(p == 5)
        def _f():
            fn(cbuf5_v)

    def _service(k):
        @pl.when(k < nck)
        def _step():
            @pl.when(k >= 3)
            def _wout():
                _on_cbuf(lax.rem(k - 3, NCB), lambda b: pltpu.make_async_copy(
                    b, _dst(k - 3), cosem).wait())
            @pl.when(k + 3 < nck)
            def _iin():
                _on_cbuf(lax.rem(k + 3, NCB), lambda b: pltpu.async_copy(
                    _src(k + 3), b, cisem))
            _on_cbuf(lax.rem(k, NCB), lambda b: pltpu.make_async_copy(
                _src(k), b, cisem).wait())
            _on_cbuf(lax.rem(k, NCB), lambda b: pltpu.async_copy(
                b, _dst(k), cosem))

    _on_cbuf(0, lambda b: pltpu.async_copy(_src(0), b, cisem))
    _on_cbuf(1, lambda b: pltpu.async_copy(_src(1), b, cisem))
    _on_cbuf(2, lambda b: pltpu.async_copy(_src(2), b, cisem))

    @pl.when(jnp.logical_not(is_last))
    def _stage_full():
        pltpu.async_copy(idxh.at[pl.ds(lo, R)], idxbuf_v, isem)

    @pl.when(is_last)
    def _stage_tail():
        pltpu.async_copy(
            idxh.at[pl.ds(lo, RLAST)], idxbuf_v.at[pl.ds(0, RLAST)], isem)

    pltpu.sync_copy(bidxh, bidx_v)

    neg1 = jnp.full((L,), -1, jnp.int32)
    def _init(i, _):
        winner_v[pl.ds(i * L, L)] = neg1
        return 0
    lax.fori_loop(0, WSZ // L, _init, 0, unroll=8)

    def _scan(g, _):
        d = bidx_v[pl.ds(g * L, L)]
        j = g * L + iota
        inr = (d >= lo) & (d < hi)
        last = plsc.scan_count(d, mask=inr)[1]
        plsc.store_scatter(
            winner_v, [jnp.where(inr, d - lo, R)], j, mask=last
        )
        @pl.when(jnp.bitwise_and(g, 7) == 0)
        def _svc():
            _service(lax.shift_right_logical(g, 3))
        return 0
    lax.fori_loop(0, B // L, _scan, 0)

    def _jlist(ci):
        return jflat_v.at[pl.ds(ci * CH, CH)]

    def _dlist(ci):
        return dflat_v.at[pl.ds(ci * CH, CH)]

    def _on_abuf(p, fn):
        @pl.when(p == 0)
        def _a():
            fn(abuf0_v)
        @pl.when(p != 0)
        def _b():
            fn(abuf1_v)

    def _compact(g, carry):
        cnt, bestv, fired = carry
        w = winner_v[pl.ds(g * L, L)]
        m = w >= 0
        dst = lo + g * L + iota
        plsc.store_compressed(jflat_v.at[pl.ds(cnt, L)], w, mask=m)
        plsc.store_compressed(dflat_v.at[pl.ds(cnt, L)], dst, mask=m)
        popc = plsc.all_reduce_population_count(m)
        npop = popc if popc.ndim == 0 else lax.squeeze(
            lax.slice(popc, (0,), (1,)), (0,))
        ncnt = cnt + npop
        can_fire = ((fired + 1) * CH <= ncnt) & (fired < NAB)
        @pl.when(can_fire)
        def _fire():
            _on_abuf(lax.rem(fired, NAB), lambda b: pltpu.async_copy(
                bemb.at[jflat_v.at[pl.ds(fired * CH, CH)]], b, gsem))
        @pl.when(jnp.bitwise_and(g, 7) == 0)
        def _svc():
            _service(B // L // 8 + lax.shift_right_logical(g, 3))
        enc = jnp.where(m, (g * L + iota) * B + w, -1)
        return ncnt, jnp.maximum(bestv, enc), fired + can_fire.astype(jnp.int32)
    cnt, bestv, nfired = lax.fori_loop(
        0, R // L, _compact, (0, jnp.full((L,), -1, jnp.int32), 0)
    )
    best = jnp.max(bestv)

    _NSVC = B // L // 8 + R // L // 8 + 1
    def _rest(k, _):
        _service(_NSVC + k)
        return 0
    lax.fori_loop(0, jnp.maximum(nck - _NSVC, 0), _rest, 0)
    _service(_NSVC - 1)

    def _cdrain(t, _):
        k = nck - 3 + t
        @pl.when(k >= 0)
        def _w():
            _on_cbuf(lax.rem(k, NCB), lambda b: pltpu.make_async_copy(
                b, _dst(k), cosem).wait())
        return 0
    lax.fori_loop(0, 3, _cdrain, 0)

    @pl.when(is_last)
    def _copy_tail():
        pltpu.sync_copy(
            memh.at[pl.ds(lo + NCKL * CC, CTAIL)], cbuf0_v.at[pl.ds(0, CTAIL)])
        pltpu.sync_copy(
            cbuf0_v.at[pl.ds(0, CTAIL)], omemh.at[pl.ds(lo + NCKL * CC, CTAIL)])

    @pl.when(jnp.logical_not(is_last))
    def _wait_full():
        pltpu.make_async_copy(idxh.at[pl.ds(lo, R)], idxbuf_v, isem).wait()

    @pl.when(is_last)
    def _wait_tail():
        pltpu.make_async_copy(
            idxh.at[pl.ds(lo, RLAST)], idxbuf_v.at[pl.ds(0, RLAST)], isem
        ).wait()

    @pl.when(cnt > 0)
    def _move():
        pad_j = jnp.full((L,), best & (B - 1), jnp.int32)
        pad_d = jnp.full((L,), lo + lax.shift_right_logical(best, 14), jnp.int32)
        def _pad(t, _):
            jflat_v[pl.ds(cnt + t * L, L)] = pad_j
            dflat_v[pl.ds(cnt + t * L, L)] = pad_d
            return 0
        lax.fori_loop(0, CH // L, _pad, 0, unroll=4)

        def _iwrite(g, _):
            v = dflat_v[pl.ds(g * L, L)]
            plsc.store_scatter(idxbuf_v, [v - lo], v)
            return 0
        lax.fori_loop(0, (cnt + L - 1) // L, _iwrite, 0)

        @pl.when(jnp.logical_not(is_last))
        def _enq_full():
            pltpu.async_copy(idxbuf_v, oidxh.at[pl.ds(lo, R)], isem)

        @pl.when(is_last)
        def _enq_tail():
            pltpu.async_copy(
                idxbuf_v.at[pl.ds(0, RLAST)], oidxh.at[pl.ds(lo, RLAST)], isem)

        nch = (cnt + CH - 1) // CH

        @pl.when(nfired == 0)
        def _fire0():
            pltpu.async_copy(bemb.at[_jlist(0)], abuf0_v, gsem)

        def _chunk(ci, _):
            p = lax.rem(ci, NAB)
            _on_abuf(p, lambda b: pltpu.make_async_copy(
                bemb.at[_jlist(ci)], b, gsem).wait())

            @pl.when(ci + 1 < nch)
            def _next():
                q = lax.rem(ci + 1, NAB)
                @pl.when(ci >= NAB - 1)
                def _free():
                    _on_abuf(q, lambda b: pltpu.make_async_copy(
                        b, omemh.at[_dlist(ci + 1 - NAB)], ssem).wait())
                @pl.when(ci + 1 >= jnp.maximum(nfired, 1))
                def _fire():
                    _on_abuf(q, lambda b: pltpu.async_copy(
                        bemb.at[_jlist(ci + 1)], b, gsem))

            _on_abuf(p, lambda b: pltpu.async_copy(
                b, omemh.at[_dlist(ci)], ssem))
            return 0
        lax.fori_loop(0, nch, _chunk, 0)

        def _drain(k, _):
            ci = jnp.maximum(nch - NAB, 0) + k
            @pl.when(ci < nch)
            def _w():
                _on_abuf(lax.rem(ci, NAB), lambda b: pltpu.make_async_copy(
                    b, omemh.at[_dlist(ci)], ssem).wait())
            return 0
        lax.fori_loop(0, jnp.minimum(nch, NAB), _drain, 0)

    @pl.when(cnt == 0)
    def _enq_empty():
        @pl.when(jnp.logical_not(is_last))
        def _e_full():
            pltpu.async_copy(idxbuf_v, oidxh.at[pl.ds(lo, R)], isem)

        @pl.when(is_last)
        def _e_tail():
            pltpu.async_copy(
                idxbuf_v.at[pl.ds(0, RLAST)], oidxh.at[pl.ds(lo, RLAST)], isem)

    @pl.when(jnp.logical_not(is_last))
    def _flush_full():
        pltpu.make_async_copy(idxbuf_v, oidxh.at[pl.ds(lo, R)], isem).wait()

    @pl.when(is_last)
    def _flush_tail():
        pltpu.make_async_copy(
            idxbuf_v.at[pl.ds(0, RLAST)], oidxh.at[pl.ds(lo, RLAST)], isem
        ).wait()


def kernel(local_memory_embeddings, local_memory_index, batch_embeddings, batch_indexes):
    return _sc_update(
        local_memory_embeddings, local_memory_index,
        batch_embeddings, batch_indexes,
    )


# speedup vs baseline: 29.0809x; 1.0323x over previous
import functools

import jax
import jax.numpy as jnp
from jax import lax
from jax.experimental import pallas as pl
from jax.experimental.pallas import tpu as pltpu
from jax.experimental.pallas import tpu_sc as plsc

N = 250000
D = 128
B = 16384
L = 16
NC = 2
NS = 16
NW = NC * NS

R = 7824
RLAST = N - (NW - 1) * R
WSZ = R + L
CH = 96
NAB = 2
FLAT = R + CH

CC = 48
NCK = R // CC
NCKL = RLAST // CC
CTAIL = RLAST - NCKL * CC
NCB = 6

_mesh = plsc.VectorSubcoreMesh(
    core_axis_name="c", subcore_axis_name="s", num_cores=NC, num_subcores=NS
)


@functools.partial(
    pl.kernel,
    out_type=(
        jax.ShapeDtypeStruct((N, D), jnp.float32),
        jax.ShapeDtypeStruct((N,), jnp.int32),
    ),
    mesh=_mesh,
    compiler_params=pltpu.CompilerParams(needs_layout_passes=False),
    scratch_types=[
        pltpu.VMEM((B,), jnp.int32),
        pltpu.VMEM((WSZ,), jnp.int32),
        pltpu.VMEM((FLAT,), jnp.int32),
        pltpu.VMEM((FLAT,), jnp.int32),
        pltpu.VMEM((CH, D), jnp.float32),
        pltpu.VMEM((CH, D), jnp.float32),
        pltpu.VMEM((CC, D), jnp.float32),
        pltpu.VMEM((CC, D), jnp.float32),
        pltpu.VMEM((CC, D), jnp.float32),
        pltpu.VMEM((CC, D), jnp.float32),
        pltpu.VMEM((CC, D), jnp.float32),
        pltpu.VMEM((CC, D), jnp.float32),
        pltpu.VMEM((R,), jnp.int32),
        pltpu.SemaphoreType.DMA,
        pltpu.SemaphoreType.DMA,
        pltpu.SemaphoreType.DMA,
        pltpu.SemaphoreType.DMA,
        pltpu.SemaphoreType.DMA,
    ],
)
def _sc_update(memh, idxh, bemb, bidxh, omemh, oidxh,
               bidx_v, winner_v, jflat_v, dflat_v, abuf0_v, abuf1_v,
               cbuf0_v, cbuf1_v, cbuf2_v, cbuf3_v, cbuf4_v, cbuf5_v, idxbuf_v,
               isem, cisem, cosem, gsem, ssem):
    wid = lax.axis_index("s") * NC + lax.axis_index("c")
    lo = wid * R
    hi = jnp.minimum(lo + R, N)
    iota = lax.broadcasted_iota(jnp.int32, (L,), 0)
    is_last = wid == NW - 1
    nck = jnp.where(is_last, NCKL, NCK)

    def _src(k):
        return memh.at[pl.ds(lo + k * CC, CC)]

    def _dst(k):
        return omemh.at[pl.ds(lo + k * CC, CC)]

    def _on_cbuf(p, fn):
        @pl.when(p == 0)
        def _a():
            fn(cbuf0_v)
        @pl.when(p == 1)
        def _b():
            fn(cbuf1_v)
        @pl.when(p == 2)
        def _c():
            fn(cbuf2_v)
        @pl.when(p == 3)
        def _d():
            fn(cbuf3_v)
        @pl.when(p == 4)
        def _e():
            fn(cbuf4_v)
        @pl.when(p == 5)
        def _f():
            fn(cbuf5_v)

    def _service(k):
        @pl.when(k < nck)
        def _step():
            @pl.when(k >= 3)
            def _wout():
                _on_cbuf(lax.rem(k - 3, NCB), lambda b: pltpu.make_async_copy(
                    b, _dst(k - 3), cosem).wait())
            @pl.when(k + 3 < nck)
            def _iin():
                _on_cbuf(lax.rem(k + 3, NCB), lambda b: pltpu.async_copy(
                    _src(k + 3), b, cisem))
            _on_cbuf(lax.rem(k, NCB), lambda b: pltpu.make_async_copy(
                _src(k), b, cisem).wait())
            _on_cbuf(lax.rem(k, NCB), lambda b: pltpu.async_copy(
                b, _dst(k), cosem))

    _on_cbuf(0, lambda b: pltpu.async_copy(_src(0), b, cisem))
    _on_cbuf(1, lambda b: pltpu.async_copy(_src(1), b, cisem))
    _on_cbuf(2, lambda b: pltpu.async_copy(_src(2), b, cisem))

    @pl.when(jnp.logical_not(is_last))
    def _stage_full():
        pltpu.async_copy(idxh.at[pl.ds(lo, R)], idxbuf_v, isem)

    @pl.when(is_last)
    def _stage_tail():
        pltpu.async_copy(
            idxh.at[pl.ds(lo, RLAST)], idxbuf_v.at[pl.ds(0, RLAST)], isem)

    pltpu.sync_copy(bidxh, bidx_v)

    neg1 = jnp.full((L,), -1, jnp.int32)
    def _init(i, _):
        winner_v[pl.ds(i * L, L)] = neg1
        return 0
    lax.fori_loop(0, WSZ // L, _init, 0, unroll=8)

    def _scan(g, _):
        d = bidx_v[pl.ds(g * L, L)]
        j = g * L + iota
        inr = (d >= lo) & (d < hi)
        last = plsc.scan_count(d, mask=inr)[1]
        plsc.store_scatter(
            winner_v, [jnp.where(inr, d - lo, R)], j, mask=last
        )
        @pl.when(jnp.bitwise_and(g, 7) == 0)
        def _svc():
            _service(lax.shift_right_logical(g, 3))
        return 0
    lax.fori_loop(0, B // L, _scan, 0)

    def _jlist(ci):
        return jflat_v.at[pl.ds(ci * CH, CH)]

    def _dlist(ci):
        return dflat_v.at[pl.ds(ci * CH, CH)]

    def _on_abuf(p, fn):
        @pl.when(p == 0)
        def _a():
            fn(abuf0_v)
        @pl.when(p != 0)
        def _b():
            fn(abuf1_v)

    def _compact(g, carry):
        cnt, bestv, fired, sfired = carry
        w = winner_v[pl.ds(g * L, L)]
        m = w >= 0
        dst = lo + g * L + iota
        plsc.store_compressed(jflat_v.at[pl.ds(cnt, L)], w, mask=m)
        plsc.store_compressed(dflat_v.at[pl.ds(cnt, L)], dst, mask=m)
        popc = plsc.all_reduce_population_count(m)
        npop = popc if popc.ndim == 0 else lax.squeeze(
            lax.slice(popc, (0,), (1,)), (0,))
        ncnt = cnt + npop
        can_fire = ((fired + 1) * CH <= ncnt) & (fired < sfired + NAB)
        @pl.when(can_fire)
        def _fire():
            @pl.when(fired >= NAB)
            def _freebuf():
                _on_abuf(lax.rem(fired, NAB), lambda b: pltpu.make_async_copy(
                    b, omemh.at[_dlist(fired - NAB)], ssem).wait())
            _on_abuf(lax.rem(fired, NAB), lambda b: pltpu.async_copy(
                bemb.at[jflat_v.at[pl.ds(fired * CH, CH)]], b, gsem))
        fired2 = fired + can_fire.astype(jnp.int32)
        can_sc = (sfired + 2 <= fired2) & (g < 479)
        @pl.when(can_sc)
        def _scat():
            _on_abuf(lax.rem(sfired, NAB), lambda b: pltpu.make_async_copy(
                bemb.at[jflat_v.at[pl.ds(sfired * CH, CH)]], b, gsem).wait())
            _on_abuf(lax.rem(sfired, NAB), lambda b: pltpu.async_copy(
                b, omemh.at[_dlist(sfired)], ssem))
        @pl.when(jnp.bitwise_and(g, 7) == 0)
        def _svc():
            _service(B // L // 8 + lax.shift_right_logical(g, 3))
        enc = jnp.where(m, (g * L + iota) * B + w, -1)
        return (ncnt, jnp.maximum(bestv, enc), fired2,
                sfired + can_sc.astype(jnp.int32))
    cnt, bestv, nfired, nsfired = lax.fori_loop(
        0, R // L, _compact, (0, jnp.full((L,), -1, jnp.int32), 0, 0)
    )
    best = jnp.max(bestv)

    _NSVC = B // L // 8 + R // L // 8 + 1
    def _rest(k, _):
        _service(_NSVC + k)
        return 0
    lax.fori_loop(0, jnp.maximum(nck - _NSVC, 0), _rest, 0)
    _service(_NSVC - 1)

    def _cdrain(t, _):
        k = nck - 3 + t
        @pl.when(k >= 0)
        def _w():
            _on_cbuf(lax.rem(k, NCB), lambda b: pltpu.make_async_copy(
                b, _dst(k), cosem).wait())
        return 0
    lax.fori_loop(0, 3, _cdrain, 0)

    @pl.when(is_last)
    def _copy_tail():
        pltpu.sync_copy(
            memh.at[pl.ds(lo + NCKL * CC, CTAIL)], cbuf0_v.at[pl.ds(0, CTAIL)])
        pltpu.sync_copy(
            cbuf0_v.at[pl.ds(0, CTAIL)], omemh.at[pl.ds(lo + NCKL * CC, CTAIL)])

    @pl.when(jnp.logical_not(is_last))
    def _wait_full():
        pltpu.make_async_copy(idxh.at[pl.ds(lo, R)], idxbuf_v, isem).wait()

    @pl.when(is_last)
    def _wait_tail():
        pltpu.make_async_copy(
            idxh.at[pl.ds(lo, RLAST)], idxbuf_v.at[pl.ds(0, RLAST)], isem
        ).wait()

    @pl.when(cnt > 0)
    def _move():
        pad_j = jnp.full((L,), best & (B - 1), jnp.int32)
        pad_d = jnp.full((L,), lo + lax.shift_right_logical(best, 14), jnp.int32)
        def _pad(t, _):
            jflat_v[pl.ds(cnt + t * L, L)] = pad_j
            dflat_v[pl.ds(cnt + t * L, L)] = pad_d
            return 0
        lax.fori_loop(0, CH // L, _pad, 0, unroll=4)

        def _iwrite(g, _):
            v = dflat_v[pl.ds(g * L, L)]
            plsc.store_scatter(idxbuf_v, [v - lo], v)
            return 0
        lax.fori_loop(0, (cnt + L - 1) // L, _iwrite, 0)

        @pl.when(jnp.logical_not(is_last))
        def _enq_full():
            pltpu.async_copy(idxbuf_v, oidxh.at[pl.ds(lo, R)], isem)

        @pl.when(is_last)
        def _enq_tail():
            pltpu.async_copy(
                idxbuf_v.at[pl.ds(0, RLAST)], oidxh.at[pl.ds(lo, RLAST)], isem)

        nch = (cnt + CH - 1) // CH

        @pl.when(nfired == 0)
        def _fire0():
            pltpu.async_copy(bemb.at[_jlist(0)], abuf0_v, gsem)

        def _chunk(ci, _):
            p = lax.rem(ci, NAB)
            _on_abuf(p, lambda b: pltpu.make_async_copy(
                bemb.at[_jlist(ci)], b, gsem).wait())

            @pl.when(ci + 1 < nch)
            def _next():
                q = lax.rem(ci + 1, NAB)
                @pl.when(ci + 1 >= jnp.maximum(nfired, 1))
                def _fire():
                    @pl.when(ci + 1 >= NAB)
                    def _free():
                        _on_abuf(q, lambda b: pltpu.make_async_copy(
                            b, omemh.at[_dlist(ci + 1 - NAB)], ssem).wait())
                    _on_abuf(q, lambda b: pltpu.async_copy(
                        bemb.at[_jlist(ci + 1)], b, gsem))

            _on_abuf(p, lambda b: pltpu.async_copy(
                b, omemh.at[_dlist(ci)], ssem))
            return 0
        lax.fori_loop(nsfired, nch, _chunk, 0)

        def _drain(k, _):
            ci = jnp.maximum(nch - NAB, 0) + k
            @pl.when(ci < nch)
            def _w():
                _on_abuf(lax.rem(ci, NAB), lambda b: pltpu.make_async_copy(
                    b, omemh.at[_dlist(ci)], ssem).wait())
            return 0
        lax.fori_loop(0, jnp.minimum(nch, NAB), _drain, 0)

    @pl.when(cnt == 0)
    def _enq_empty():
        @pl.when(jnp.logical_not(is_last))
        def _e_full():
            pltpu.async_copy(idxbuf_v, oidxh.at[pl.ds(lo, R)], isem)

        @pl.when(is_last)
        def _e_tail():
            pltpu.async_copy(
                idxbuf_v.at[pl.ds(0, RLAST)], oidxh.at[pl.ds(lo, RLAST)], isem)

    @pl.when(jnp.logical_not(is_last))
    def _flush_full():
        pltpu.make_async_copy(idxbuf_v, oidxh.at[pl.ds(lo, R)], isem).wait()

    @pl.when(is_last)
    def _flush_tail():
        pltpu.make_async_copy(
            idxbuf_v.at[pl.ds(0, RLAST)], oidxh.at[pl.ds(lo, RLAST)], isem
        ).wait()


def kernel(local_memory_embeddings, local_memory_index, batch_embeddings, batch_indexes):
    return _sc_update(
        local_memory_embeddings, local_memory_index,
        batch_embeddings, batch_indexes,
    )
